# Initial kernel scaffold; baseline (speedup 1.0000x reference)
import jax
import jax.numpy as jnp
from jax.experimental import pallas as pl


def kernel(inputs, adj, hidden_state, W_gate_0, b_gate_0, W_cand_0, b_cand_0, W_gate_1, b_gate_1, W_cand_1, b_cand_1):
    raise NotImplementedError("write your pallas kernel here")


# speedup vs baseline: 1.5227x; 1.5227x over previous
import jax
import jax.numpy as jnp
from jax.experimental import pallas as pl

N = 1024
B = 32
U = 64
NB = N * B
BLK = 256
RB = 4096
F32 = jnp.float32
BF16 = jnp.bfloat16


def _d1_pair_body(a_ref, x0_ref, h0_ref, x1_ref, h1_ref):
    a = a_ref[...]
    x1_ref[...] = jnp.dot(a, x0_ref[...], preferred_element_type=F32).astype(BF16)
    h1_ref[...] = jnp.dot(a, h0_ref[...], preferred_element_type=F32).astype(BF16)


def _diffuse1_pair(adj, x0, h0):
    fx, fh = x0.shape[1], h0.shape[1]
    return pl.pallas_call(
        _d1_pair_body,
        grid=(N // BLK,),
        in_specs=[
            pl.BlockSpec((BLK, N), lambda i: (i, 0)),
            pl.BlockSpec((N, fx), lambda i: (0, 0)),
            pl.BlockSpec((N, fh), lambda i: (0, 0)),
        ],
        out_specs=[
            pl.BlockSpec((BLK, fx), lambda i: (i, 0)),
            pl.BlockSpec((BLK, fh), lambda i: (i, 0)),
        ],
        out_shape=[
            jax.ShapeDtypeStruct((N, fx), BF16),
            jax.ShapeDtypeStruct((N, fh), BF16),
        ],
    )(adj, x0, h0)


def _d2_pair_body(a_ref, x0_ref, x1_ref, h0_ref, h1_ref, x2_ref, h2_ref):
    a = a_ref[...]
    x2 = 2.0 * jnp.dot(a, x1_ref[...], preferred_element_type=F32)
    x2_ref[...] = (x2 - x0_ref[...].astype(F32)).astype(BF16)
    h2 = 2.0 * jnp.dot(a, h1_ref[...], preferred_element_type=F32)
    h2_ref[...] = (h2 - h0_ref[...].astype(F32)).astype(BF16)


def _diffuse2_pair(adj, x0, x1, h0, h1):
    fx, fh = x0.shape[1], h0.shape[1]
    return pl.pallas_call(
        _d2_pair_body,
        grid=(N // BLK,),
        in_specs=[
            pl.BlockSpec((BLK, N), lambda i: (i, 0)),
            pl.BlockSpec((BLK, fx), lambda i: (i, 0)),
            pl.BlockSpec((N, fx), lambda i: (0, 0)),
            pl.BlockSpec((BLK, fh), lambda i: (i, 0)),
            pl.BlockSpec((N, fh), lambda i: (0, 0)),
        ],
        out_specs=[
            pl.BlockSpec((BLK, fx), lambda i: (i, 0)),
            pl.BlockSpec((BLK, fh), lambda i: (i, 0)),
        ],
        out_shape=[
            jax.ShapeDtypeStruct((N, fx), BF16),
            jax.ShapeDtypeStruct((N, fh), BF16),
        ],
    )(adj, x0, x1, h0, h1)


def _d1_single_body(a_ref, s0_ref, s1_ref):
    s1_ref[...] = jnp.dot(a_ref[...], s0_ref[...], preferred_element_type=F32).astype(BF16)


def _diffuse1_single(adj, s0):
    f = s0.shape[1]
    return pl.pallas_call(
        _d1_single_body,
        grid=(N // BLK,),
        in_specs=[
            pl.BlockSpec((BLK, N), lambda i: (i, 0)),
            pl.BlockSpec((N, f), lambda i: (0, 0)),
        ],
        out_specs=pl.BlockSpec((BLK, f), lambda i: (i, 0)),
        out_shape=jax.ShapeDtypeStruct((N, f), BF16),
    )(adj, s0)


def _d2_single_body(a_ref, s0_ref, s1_ref, s2_ref):
    s2 = 2.0 * jnp.dot(a_ref[...], s1_ref[...], preferred_element_type=F32)
    s2_ref[...] = (s2 - s0_ref[...].astype(F32)).astype(BF16)


def _diffuse2_single(adj, s0, s1):
    f = s0.shape[1]
    return pl.pallas_call(
        _d2_single_body,
        grid=(N // BLK,),
        in_specs=[
            pl.BlockSpec((BLK, N), lambda i: (i, 0)),
            pl.BlockSpec((BLK, f), lambda i: (i, 0)),
            pl.BlockSpec((N, f), lambda i: (0, 0)),
        ],
        out_specs=pl.BlockSpec((BLK, f), lambda i: (i, 0)),
        out_shape=jax.ShapeDtypeStruct((N, f), BF16),
    )(adj, s0, s1)


def _gate_body(x0_ref, x1_ref, x2_ref, h0_ref, h1_ref, h2_ref, hf_ref,
               wx_ref, wh_ref, bg_ref, s0_ref, u_ref):
    g = jnp.dot(x0_ref[...], wx_ref[0], preferred_element_type=F32)
    g += jnp.dot(x1_ref[...], wx_ref[1], preferred_element_type=F32)
    g += jnp.dot(x2_ref[...], wx_ref[2], preferred_element_type=F32)
    g += jnp.dot(h0_ref[...], wh_ref[0], preferred_element_type=F32)
    g += jnp.dot(h1_ref[...], wh_ref[1], preferred_element_type=F32)
    g += jnp.dot(h2_ref[...], wh_ref[2], preferred_element_type=F32)
    g = jax.nn.sigmoid(g + bg_ref[...])
    r = g[:, :U]
    u = g[:, U:]
    s0_ref[...] = (r * hf_ref[...]).astype(BF16)
    u_ref[...] = u


def _gate(x0, x1, x2, h0, h1, h2, hf, wx, wh, bg):
    cin = x0.shape[1]
    return pl.pallas_call(
        _gate_body,
        grid=(NB // RB,),
        in_specs=[
            pl.BlockSpec((RB, cin), lambda i: (i, 0)),
            pl.BlockSpec((RB, cin), lambda i: (i, 0)),
            pl.BlockSpec((RB, cin), lambda i: (i, 0)),
            pl.BlockSpec((RB, U), lambda i: (i, 0)),
            pl.BlockSpec((RB, U), lambda i: (i, 0)),
            pl.BlockSpec((RB, U), lambda i: (i, 0)),
            pl.BlockSpec((RB, U), lambda i: (i, 0)),
            pl.BlockSpec((3, cin, 2 * U), lambda i: (0, 0, 0)),
            pl.BlockSpec((3, U, 2 * U), lambda i: (0, 0, 0)),
            pl.BlockSpec((1, 2 * U), lambda i: (0, 0)),
        ],
        out_specs=[
            pl.BlockSpec((RB, U), lambda i: (i, 0)),
            pl.BlockSpec((RB, U), lambda i: (i, 0)),
        ],
        out_shape=[
            jax.ShapeDtypeStruct((NB, U), BF16),
            jax.ShapeDtypeStruct((NB, U), F32),
        ],
    )(x0, x1, x2, h0, h1, h2, hf, wx, wh, bg)


def _cand_body(x0_ref, x1_ref, x2_ref, s0_ref, s1_ref, s2_ref, u_ref, hf_ref,
               wx_ref, ws_ref, bc_ref, hn_ref, hnb_ref):
    c = jnp.dot(x0_ref[...], wx_ref[0], preferred_element_type=F32)
    c += jnp.dot(x1_ref[...], wx_ref[1], preferred_element_type=F32)
    c += jnp.dot(x2_ref[...], wx_ref[2], preferred_element_type=F32)
    c += jnp.dot(s0_ref[...], ws_ref[0], preferred_element_type=F32)
    c += jnp.dot(s1_ref[...], ws_ref[1], preferred_element_type=F32)
    c += jnp.dot(s2_ref[...], ws_ref[2], preferred_element_type=F32)
    c = jnp.tanh(c + bc_ref[...])
    u = u_ref[...]
    hn = u * hf_ref[...] + (1.0 - u) * c
    hn_ref[...] = hn
    hnb_ref[...] = hn.astype(BF16)


def _cand(x0, x1, x2, s0, s1, s2, u, hf, wx, ws, bc):
    cin = x0.shape[1]
    return pl.pallas_call(
        _cand_body,
        grid=(NB // RB,),
        in_specs=[
            pl.BlockSpec((RB, cin), lambda i: (i, 0)),
            pl.BlockSpec((RB, cin), lambda i: (i, 0)),
            pl.BlockSpec((RB, cin), lambda i: (i, 0)),
            pl.BlockSpec((RB, U), lambda i: (i, 0)),
            pl.BlockSpec((RB, U), lambda i: (i, 0)),
            pl.BlockSpec((RB, U), lambda i: (i, 0)),
            pl.BlockSpec((RB, U), lambda i: (i, 0)),
            pl.BlockSpec((RB, U), lambda i: (i, 0)),
            pl.BlockSpec((3, cin, U), lambda i: (0, 0, 0)),
            pl.BlockSpec((3, U, U), lambda i: (0, 0, 0)),
            pl.BlockSpec((1, U), lambda i: (0, 0)),
        ],
        out_specs=[
            pl.BlockSpec((RB, U), lambda i: (i, 0)),
            pl.BlockSpec((RB, U), lambda i: (i, 0)),
        ],
        out_shape=[
            jax.ShapeDtypeStruct((NB, U), F32),
            jax.ShapeDtypeStruct((NB, U), BF16),
        ],
    )(x0, x1, x2, s0, s1, s2, u, hf, wx, ws, bc)


def _layer(adj_bf, x0_bf, h_f32, h_bf, wgx, wgh, bg, wcx, wch, bc):
    cin = x0_bf.shape[1]
    x0_2d = x0_bf.reshape(N, B * cin)
    h0_2d = h_bf.reshape(N, B * U)
    x1_2d, h1_2d = _diffuse1_pair(adj_bf, x0_2d, h0_2d)
    x2_2d, h2_2d = _diffuse2_pair(adj_bf, x0_2d, x1_2d, h0_2d, h1_2d)
    s0, u = _gate(
        x0_bf, x1_2d.reshape(NB, cin), x2_2d.reshape(NB, cin),
        h_bf, h1_2d.reshape(NB, U), h2_2d.reshape(NB, U),
        h_f32, wgx, wgh, bg,
    )
    s0_2d = s0.reshape(N, B * U)
    s1_2d = _diffuse1_single(adj_bf, s0_2d)
    s2_2d = _diffuse2_single(adj_bf, s0_2d, s1_2d)
    return _cand(
        x0_bf, x1_2d.reshape(NB, cin), x2_2d.reshape(NB, cin),
        s0, s1_2d.reshape(NB, U), s2_2d.reshape(NB, U),
        u, h_f32, wcx, wch, bc,
    )


def _prep_w(w, cin):
    o = w.shape[1]
    wr = w.reshape(cin + U, 3, o).transpose(1, 0, 2).astype(BF16)
    return wr[:, :cin, :], wr[:, cin:, :]


def kernel(inputs, adj, hidden_state,
           W_gate_0, b_gate_0, W_cand_0, b_cand_0,
           W_gate_1, b_gate_1, W_cand_1, b_cand_1):
    adj_bf = adj.astype(BF16)
    x0 = inputs.reshape(B, N, 2).transpose(1, 0, 2).reshape(NB, 2).astype(BF16)
    h0_nm = hidden_state[0].reshape(B, N, U).transpose(1, 0, 2).reshape(NB, U)
    h1_nm = hidden_state[1].reshape(B, N, U).transpose(1, 0, 2).reshape(NB, U)

    wg0x, wg0h = _prep_w(W_gate_0, 2)
    wc0x, wc0h = _prep_w(W_cand_0, 2)
    wg1x, wg1h = _prep_w(W_gate_1, U)
    wc1x, wc1h = _prep_w(W_cand_1, U)
    bg0 = b_gate_0.reshape(1, 2 * U)
    bc0 = b_cand_0.reshape(1, U)
    bg1 = b_gate_1.reshape(1, 2 * U)
    bc1 = b_cand_1.reshape(1, U)

    h0f, h0b = _layer(adj_bf, x0, h0_nm, h0_nm.astype(BF16),
                      wg0x, wg0h, bg0, wc0x, wc0h, bc0)
    h1f, _ = _layer(adj_bf, h0b, h1_nm, h1_nm.astype(BF16),
                    wg1x, wg1h, bg1, wc1x, wc1h, bc1)

    out0 = h0f.reshape(N, B, U).transpose(1, 0, 2).reshape(B, N * U)
    out1 = h1f.reshape(N, B, U).transpose(1, 0, 2).reshape(B, N * U)
    return (out1, jnp.stack([out0, out1]))


# speedup vs baseline: 1.5284x; 1.0038x over previous
import jax
import jax.numpy as jnp
from jax.experimental import pallas as pl

N = 1024
B = 32
U = 64
NB = N * B
BLK = 256
RB = 4096
F32 = jnp.float32
BF16 = jnp.bfloat16


def _d1_pair_body(a_ref, x0_ref, h0_ref, x1_ref, h1_ref):
    a = a_ref[...]
    x1_ref[...] = jnp.dot(a, x0_ref[...], preferred_element_type=F32).astype(BF16)
    h1_ref[...] = jnp.dot(a, h0_ref[...], preferred_element_type=F32).astype(BF16)


def _diffuse1_pair(adj, x0, h0):
    fx, fh = x0.shape[1], h0.shape[1]
    return pl.pallas_call(
        _d1_pair_body,
        grid=(N // BLK,),
        in_specs=[
            pl.BlockSpec((BLK, N), lambda i: (i, 0)),
            pl.BlockSpec((N, fx), lambda i: (0, 0)),
            pl.BlockSpec((N, fh), lambda i: (0, 0)),
        ],
        out_specs=[
            pl.BlockSpec((BLK, fx), lambda i: (i, 0)),
            pl.BlockSpec((BLK, fh), lambda i: (i, 0)),
        ],
        out_shape=[
            jax.ShapeDtypeStruct((N, fx), BF16),
            jax.ShapeDtypeStruct((N, fh), BF16),
        ],
    )(adj, x0, h0)


def _d2_pair_body(a_ref, x0_ref, x1_ref, h0_ref, h1_ref, x2_ref, h2_ref):
    a = a_ref[...]
    x2 = 2.0 * jnp.dot(a, x1_ref[...], preferred_element_type=F32)
    x2_ref[...] = (x2 - x0_ref[...].astype(F32)).astype(BF16)
    h2 = 2.0 * jnp.dot(a, h1_ref[...], preferred_element_type=F32)
    h2_ref[...] = (h2 - h0_ref[...].astype(F32)).astype(BF16)


def _diffuse2_pair(adj, x0, x1, h0, h1):
    fx, fh = x0.shape[1], h0.shape[1]
    return pl.pallas_call(
        _d2_pair_body,
        grid=(N // BLK,),
        in_specs=[
            pl.BlockSpec((BLK, N), lambda i: (i, 0)),
            pl.BlockSpec((BLK, fx), lambda i: (i, 0)),
            pl.BlockSpec((N, fx), lambda i: (0, 0)),
            pl.BlockSpec((BLK, fh), lambda i: (i, 0)),
            pl.BlockSpec((N, fh), lambda i: (0, 0)),
        ],
        out_specs=[
            pl.BlockSpec((BLK, fx), lambda i: (i, 0)),
            pl.BlockSpec((BLK, fh), lambda i: (i, 0)),
        ],
        out_shape=[
            jax.ShapeDtypeStruct((N, fx), BF16),
            jax.ShapeDtypeStruct((N, fh), BF16),
        ],
    )(adj, x0, x1, h0, h1)


def _d1_single_body(a_ref, s0_ref, s1_ref):
    s1_ref[...] = jnp.dot(a_ref[...], s0_ref[...], preferred_element_type=F32).astype(BF16)


def _diffuse1_single(adj, s0):
    f = s0.shape[1]
    return pl.pallas_call(
        _d1_single_body,
        grid=(N // BLK,),
        in_specs=[
            pl.BlockSpec((BLK, N), lambda i: (i, 0)),
            pl.BlockSpec((N, f), lambda i: (0, 0)),
        ],
        out_specs=pl.BlockSpec((BLK, f), lambda i: (i, 0)),
        out_shape=jax.ShapeDtypeStruct((N, f), BF16),
    )(adj, s0)


def _d2_single_body(a_ref, s0_ref, s1_ref, s2_ref):
    s2 = 2.0 * jnp.dot(a_ref[...], s1_ref[...], preferred_element_type=F32)
    s2_ref[...] = (s2 - s0_ref[...].astype(F32)).astype(BF16)


def _diffuse2_single(adj, s0, s1):
    f = s0.shape[1]
    return pl.pallas_call(
        _d2_single_body,
        grid=(N // BLK,),
        in_specs=[
            pl.BlockSpec((BLK, N), lambda i: (i, 0)),
            pl.BlockSpec((BLK, f), lambda i: (i, 0)),
            pl.BlockSpec((N, f), lambda i: (0, 0)),
        ],
        out_specs=pl.BlockSpec((BLK, f), lambda i: (i, 0)),
        out_shape=jax.ShapeDtypeStruct((N, f), BF16),
    )(adj, s0, s1)


def _prep_body(inp_ref, h0_ref, h1_ref,
               x0_ref, h0b_ref, h0f_ref, h1b_ref, h1f_ref):
    x = jnp.transpose(inp_ref[...], (1, 0, 2))
    x0_ref[...] = x.reshape(-1, 2).astype(BF16)
    h0 = jnp.transpose(h0_ref[...], (1, 0, 2)).reshape(-1, U)
    h0f_ref[...] = h0
    h0b_ref[...] = h0.astype(BF16)
    h1 = jnp.transpose(h1_ref[...], (1, 0, 2)).reshape(-1, U)
    h1f_ref[...] = h1
    h1b_ref[...] = h1.astype(BF16)


def _prep(inp, h0, h1):
    blkp = 128
    return pl.pallas_call(
        _prep_body,
        grid=(N // blkp,),
        in_specs=[
            pl.BlockSpec((B, blkp, 2), lambda i: (0, i, 0)),
            pl.BlockSpec((B, blkp, U), lambda i: (0, i, 0)),
            pl.BlockSpec((B, blkp, U), lambda i: (0, i, 0)),
        ],
        out_specs=[
            pl.BlockSpec((blkp * B, 2), lambda i: (i, 0)),
            pl.BlockSpec((blkp * B, U), lambda i: (i, 0)),
            pl.BlockSpec((blkp * B, U), lambda i: (i, 0)),
            pl.BlockSpec((blkp * B, U), lambda i: (i, 0)),
            pl.BlockSpec((blkp * B, U), lambda i: (i, 0)),
        ],
        out_shape=[
            jax.ShapeDtypeStruct((NB, 2), BF16),
            jax.ShapeDtypeStruct((NB, U), BF16),
            jax.ShapeDtypeStruct((NB, U), F32),
            jax.ShapeDtypeStruct((NB, U), BF16),
            jax.ShapeDtypeStruct((NB, U), F32),
        ],
    )(inp, h0, h1)


def _gate_body(x0_ref, x1_ref, x2_ref, h0_ref, h1_ref, h2_ref, hf_ref,
               wx_ref, wh_ref, bg_ref, s0_ref, u_ref):
    g = jnp.dot(x0_ref[...], wx_ref[0], preferred_element_type=F32)
    g += jnp.dot(x1_ref[...], wx_ref[1], preferred_element_type=F32)
    g += jnp.dot(x2_ref[...], wx_ref[2], preferred_element_type=F32)
    g += jnp.dot(h0_ref[...], wh_ref[0], preferred_element_type=F32)
    g += jnp.dot(h1_ref[...], wh_ref[1], preferred_element_type=F32)
    g += jnp.dot(h2_ref[...], wh_ref[2], preferred_element_type=F32)
    g = jax.nn.sigmoid(g + bg_ref[...])
    r = g[:, :U]
    u = g[:, U:]
    s0_ref[...] = (r * hf_ref[...]).astype(BF16)
    u_ref[...] = u


def _gate(x0, x1, x2, h0, h1, h2, hf, wx, wh, bg):
    cin = x0.shape[1]
    return pl.pallas_call(
        _gate_body,
        grid=(NB // RB,),
        in_specs=[
            pl.BlockSpec((RB, cin), lambda i: (i, 0)),
            pl.BlockSpec((RB, cin), lambda i: (i, 0)),
            pl.BlockSpec((RB, cin), lambda i: (i, 0)),
            pl.BlockSpec((RB, U), lambda i: (i, 0)),
            pl.BlockSpec((RB, U), lambda i: (i, 0)),
            pl.BlockSpec((RB, U), lambda i: (i, 0)),
            pl.BlockSpec((RB, U), lambda i: (i, 0)),
            pl.BlockSpec((3, cin, 2 * U), lambda i: (0, 0, 0)),
            pl.BlockSpec((3, U, 2 * U), lambda i: (0, 0, 0)),
            pl.BlockSpec((1, 2 * U), lambda i: (0, 0)),
        ],
        out_specs=[
            pl.BlockSpec((RB, U), lambda i: (i, 0)),
            pl.BlockSpec((RB, U), lambda i: (i, 0)),
        ],
        out_shape=[
            jax.ShapeDtypeStruct((NB, U), BF16),
            jax.ShapeDtypeStruct((NB, U), F32),
        ],
    )(x0, x1, x2, h0, h1, h2, hf, wx, wh, bg)


def _cand_body(x0_ref, x1_ref, x2_ref, s0_ref, s1_ref, s2_ref, u_ref, hf_ref,
               wx_ref, ws_ref, bc_ref, hn_ref, hnb_ref):
    c = jnp.dot(x0_ref[...], wx_ref[0], preferred_element_type=F32)
    c += jnp.dot(x1_ref[...], wx_ref[1], preferred_element_type=F32)
    c += jnp.dot(x2_ref[...], wx_ref[2], preferred_element_type=F32)
    c += jnp.dot(s0_ref[...], ws_ref[0], preferred_element_type=F32)
    c += jnp.dot(s1_ref[...], ws_ref[1], preferred_element_type=F32)
    c += jnp.dot(s2_ref[...], ws_ref[2], preferred_element_type=F32)
    c = jnp.tanh(c + bc_ref[...])
    u = u_ref[...]
    hn = u * hf_ref[...] + (1.0 - u) * c
    hn_ref[...] = jnp.transpose(hn.reshape(RB // B, B, U), (1, 0, 2))
    hnb_ref[...] = hn.astype(BF16)


def _cand(x0, x1, x2, s0, s1, s2, u, hf, wx, ws, bc):
    cin = x0.shape[1]
    return pl.pallas_call(
        _cand_body,
        grid=(NB // RB,),
        in_specs=[
            pl.BlockSpec((RB, cin), lambda i: (i, 0)),
            pl.BlockSpec((RB, cin), lambda i: (i, 0)),
            pl.BlockSpec((RB, cin), lambda i: (i, 0)),
            pl.BlockSpec((RB, U), lambda i: (i, 0)),
            pl.BlockSpec((RB, U), lambda i: (i, 0)),
            pl.BlockSpec((RB, U), lambda i: (i, 0)),
            pl.BlockSpec((RB, U), lambda i: (i, 0)),
            pl.BlockSpec((RB, U), lambda i: (i, 0)),
            pl.BlockSpec((3, cin, U), lambda i: (0, 0, 0)),
            pl.BlockSpec((3, U, U), lambda i: (0, 0, 0)),
            pl.BlockSpec((1, U), lambda i: (0, 0)),
        ],
        out_specs=[
            pl.BlockSpec((B, RB // B, U), lambda i: (0, i, 0)),
            pl.BlockSpec((RB, U), lambda i: (i, 0)),
        ],
        out_shape=[
            jax.ShapeDtypeStruct((B, N, U), F32),
            jax.ShapeDtypeStruct((NB, U), BF16),
        ],
    )(x0, x1, x2, s0, s1, s2, u, hf, wx, ws, bc)


def _layer(adj_bf, x0_bf, h_f32, h_bf, wgx, wgh, bg, wcx, wch, bc):
    cin = x0_bf.shape[1]
    x0_2d = x0_bf.reshape(N, B * cin)
    h0_2d = h_bf.reshape(N, B * U)
    x1_2d, h1_2d = _diffuse1_pair(adj_bf, x0_2d, h0_2d)
    x2_2d, h2_2d = _diffuse2_pair(adj_bf, x0_2d, x1_2d, h0_2d, h1_2d)
    s0, u = _gate(
        x0_bf, x1_2d.reshape(NB, cin), x2_2d.reshape(NB, cin),
        h_bf, h1_2d.reshape(NB, U), h2_2d.reshape(NB, U),
        h_f32, wgx, wgh, bg,
    )
    s0_2d = s0.reshape(N, B * U)
    s1_2d = _diffuse1_single(adj_bf, s0_2d)
    s2_2d = _diffuse2_single(adj_bf, s0_2d, s1_2d)
    return _cand(
        x0_bf, x1_2d.reshape(NB, cin), x2_2d.reshape(NB, cin),
        s0, s1_2d.reshape(NB, U), s2_2d.reshape(NB, U),
        u, h_f32, wcx, wch, bc,
    )


def _prep_w(w, cin):
    o = w.shape[1]
    wr = w.reshape(cin + U, 3, o).transpose(1, 0, 2).astype(BF16)
    return wr[:, :cin, :], wr[:, cin:, :]


def kernel(inputs, adj, hidden_state,
           W_gate_0, b_gate_0, W_cand_0, b_cand_0,
           W_gate_1, b_gate_1, W_cand_1, b_cand_1):
    adj_bf = adj.astype(BF16)
    x0, h0_bf, h0_f, h1_bf, h1_f = _prep(
        inputs.reshape(B, N, 2),
        hidden_state[0].reshape(B, N, U),
        hidden_state[1].reshape(B, N, U),
    )

    wg0x, wg0h = _prep_w(W_gate_0, 2)
    wc0x, wc0h = _prep_w(W_cand_0, 2)
    wg1x, wg1h = _prep_w(W_gate_1, U)
    wc1x, wc1h = _prep_w(W_cand_1, U)
    bg0 = b_gate_0.reshape(1, 2 * U)
    bc0 = b_cand_0.reshape(1, U)
    bg1 = b_gate_1.reshape(1, 2 * U)
    bc1 = b_cand_1.reshape(1, U)

    h0bm, h0b = _layer(adj_bf, x0, h0_f, h0_bf,
                       wg0x, wg0h, bg0, wc0x, wc0h, bc0)
    h1bm, _ = _layer(adj_bf, h0b, h1_f, h1_bf,
                     wg1x, wg1h, bg1, wc1x, wc1h, bc1)

    out0 = h0bm.reshape(B, N * U)
    out1 = h1bm.reshape(B, N * U)
    return (out1, jnp.stack([out0, out1]))


# speedup vs baseline: 2.9735x; 1.9455x over previous
import jax
import jax.numpy as jnp
from jax.experimental import pallas as pl

N = 1024
B = 32
U = 64
C = 2 * U
NB = N * B
BLK = 256
F32 = jnp.float32
BF16 = jnp.bfloat16


def _prep_body(x_ref, hs_ref, y0_ref, h1p_ref):
    x = x_ref[...]
    h0 = jnp.transpose(hs_ref[0], (1, 0, 2))
    h1 = jnp.transpose(hs_ref[1], (1, 0, 2))
    blkp = x.shape[0]
    y0 = jnp.concatenate([x, h0.astype(BF16)], axis=-1)
    y0_ref[...] = y0.reshape(blkp, B * C)
    h1p = jnp.concatenate([jnp.zeros_like(x), h1.astype(BF16)], axis=-1)
    h1p_ref[...] = h1p.reshape(blkp, B * C)


def _prep(xin, hs4):
    blkp = 128
    return pl.pallas_call(
        _prep_body,
        grid=(N // blkp,),
        in_specs=[
            pl.BlockSpec((blkp, B, U), lambda i: (i, 0, 0)),
            pl.BlockSpec((2, B, blkp, U), lambda i: (0, 0, i, 0)),
        ],
        out_specs=[
            pl.BlockSpec((blkp, B * C), lambda i: (i, 0)),
            pl.BlockSpec((blkp, B * C), lambda i: (i, 0)),
        ],
        out_shape=[
            jax.ShapeDtypeStruct((N, B * C), BF16),
            jax.ShapeDtypeStruct((N, B * C), BF16),
        ],
    )(xin, hs4)


def _d1_body(a_ref, y0_ref, y1_ref):
    y1_ref[...] = jnp.dot(a_ref[...], y0_ref[...], preferred_element_type=F32).astype(BF16)


def _d1(adj, y0):
    return pl.pallas_call(
        _d1_body,
        grid=(N // BLK,),
        in_specs=[
            pl.BlockSpec((BLK, N), lambda i: (i, 0)),
            pl.BlockSpec((N, B * C), lambda i: (0, 0)),
        ],
        out_specs=pl.BlockSpec((BLK, B * C), lambda i: (i, 0)),
        out_shape=jax.ShapeDtypeStruct((N, B * C), BF16),
    )(adj, y0)


def _gate_body(a_ref, y0_ref, y1_ref, wg_ref, bg_ref, y2_ref, s0_ref, g_ref):
    a = a_ref[...]
    y0b = y0_ref[...]
    y1b = y1_ref[pl.ds(pl.program_id(0) * BLK, BLK), :]
    y2 = 2.0 * jnp.dot(a, y1_ref[...], preferred_element_type=F32) - y0b.astype(F32)
    y2bf = y2.astype(BF16)
    y2_ref[...] = y2bf

    y0r = y0b.reshape(BLK * B, C)
    y1r = y1b.reshape(BLK * B, C)
    y2r = y2bf.reshape(BLK * B, C)
    g = jnp.dot(y0r, wg_ref[0], preferred_element_type=F32)
    g += jnp.dot(y1r, wg_ref[1], preferred_element_type=F32)
    g += jnp.dot(y2r, wg_ref[2], preferred_element_type=F32)
    g = jax.nn.sigmoid(g + bg_ref[...])
    r = g[:, :U]
    h = y0r[:, U:].astype(F32)
    s0 = (r * h).astype(BF16)
    z = jnp.zeros_like(s0)
    s0_ref[...] = jnp.concatenate([s0, z], axis=-1).reshape(BLK, B * C)
    g_ref[...] = g.astype(BF16).reshape(BLK, B * C)


def _gate(adj, y0, y1, wg, bg):
    return pl.pallas_call(
        _gate_body,
        grid=(N // BLK,),
        in_specs=[
            pl.BlockSpec((BLK, N), lambda i: (i, 0)),
            pl.BlockSpec((BLK, B * C), lambda i: (i, 0)),
            pl.BlockSpec((N, B * C), lambda i: (0, 0)),
            pl.BlockSpec((3, C, C), lambda i: (0, 0, 0)),
            pl.BlockSpec((1, C), lambda i: (0, 0)),
        ],
        out_specs=[
            pl.BlockSpec((BLK, B * C), lambda i: (i, 0)),
            pl.BlockSpec((BLK, B * C), lambda i: (i, 0)),
            pl.BlockSpec((BLK, B * C), lambda i: (i, 0)),
        ],
        out_shape=[
            jax.ShapeDtypeStruct((N, B * C), BF16),
            jax.ShapeDtypeStruct((N, B * C), BF16),
            jax.ShapeDtypeStruct((N, B * C), BF16),
        ],
    )(adj, y0, y1, wg, bg)


def _cand_body(a_ref, y0_ref, y1_ref, y2_ref, s0_ref, s1_ref, g_ref, h1p_ref,
               wcx_ref, wcs_ref, bc_ref, y0n_ref, hbm_ref):
    a = a_ref[...]
    s0b = s0_ref[...]
    s1b = s1_ref[pl.ds(pl.program_id(0) * BLK, BLK), :]
    s2 = 2.0 * jnp.dot(a, s1_ref[...], preferred_element_type=F32) - s0b.astype(F32)

    y0r = y0_ref[...].reshape(BLK * B, C)
    y1r = y1_ref[...].reshape(BLK * B, C)
    y2r = y2_ref[...].reshape(BLK * B, C)
    s0r = s0b.reshape(BLK * B, C)
    s1r = s1b.reshape(BLK * B, C)
    s2r = s2.astype(BF16).reshape(BLK * B, C)
    c = jnp.dot(y0r, wcx_ref[0], preferred_element_type=F32)
    c += jnp.dot(y1r, wcx_ref[1], preferred_element_type=F32)
    c += jnp.dot(y2r, wcx_ref[2], preferred_element_type=F32)
    c += jnp.dot(s0r, wcs_ref[0], preferred_element_type=F32)
    c += jnp.dot(s1r, wcs_ref[1], preferred_element_type=F32)
    c += jnp.dot(s2r, wcs_ref[2], preferred_element_type=F32)
    c = jnp.tanh(c + bc_ref[...])
    gr = g_ref[...].reshape(BLK * B, C)
    u = gr[:, U:].astype(F32)
    hx = y0r[:, U:].astype(F32)
    hn = u * hx + (1.0 - u) * c

    h1pr = h1p_ref[...].reshape(BLK * B, C)
    y0n = jnp.concatenate([hn.astype(BF16), h1pr[:, U:]], axis=-1)
    y0n_ref[...] = y0n.reshape(BLK, B * C)

    hp = hn.reshape(BLK // 2, 2, B, U)
    cc = jnp.concatenate([hp[:, 0], hp[:, 1]], axis=-1)
    hbm_ref[...] = jnp.transpose(cc, (1, 0, 2)).reshape(B, BLK * U)


def _cand(adj, y0, y1, y2, s0, s1, g, h1p, wcx, wcs, bc):
    return pl.pallas_call(
        _cand_body,
        grid=(N // BLK,),
        in_specs=[
            pl.BlockSpec((BLK, N), lambda i: (i, 0)),
            pl.BlockSpec((BLK, B * C), lambda i: (i, 0)),
            pl.BlockSpec((BLK, B * C), lambda i: (i, 0)),
            pl.BlockSpec((BLK, B * C), lambda i: (i, 0)),
            pl.BlockSpec((BLK, B * C), lambda i: (i, 0)),
            pl.BlockSpec((N, B * C), lambda i: (0, 0)),
            pl.BlockSpec((BLK, B * C), lambda i: (i, 0)),
            pl.BlockSpec((BLK, B * C), lambda i: (i, 0)),
            pl.BlockSpec((3, C, U), lambda i: (0, 0, 0)),
            pl.BlockSpec((3, C, U), lambda i: (0, 0, 0)),
            pl.BlockSpec((1, U), lambda i: (0, 0)),
        ],
        out_specs=[
            pl.BlockSpec((BLK, B * C), lambda i: (i, 0)),
            pl.BlockSpec((B, BLK * U), lambda i: (0, i)),
        ],
        out_shape=[
            jax.ShapeDtypeStruct((N, B * C), BF16),
            jax.ShapeDtypeStruct((B, N * U), F32),
        ],
    )(adj, y0, y1, y2, s0, s1, g, h1p, wcx, wcs, bc)


def _layer(adj_bf, y0, h1p, wg, bg, wcx, wcs, bc):
    y1 = _d1(adj_bf, y0)
    y2, s0, g = _gate(adj_bf, y0, y1, wg, bg)
    s1 = _d1(adj_bf, s0)
    return _cand(adj_bf, y0, y1, y2, s0, s1, g, h1p, wcx, wcs, bc)


def _prep_w(w, cin):
    o = w.shape[1]
    wr = w.reshape(cin + U, 3, o).transpose(1, 0, 2)
    wx = wr[:, :cin, :]
    wh = wr[:, cin:, :]
    pad = jnp.zeros((3, U - cin, o), w.dtype)
    wxp = jnp.concatenate([wx, pad], axis=1)
    return wxp, wh


def kernel(inputs, adj, hidden_state,
           W_gate_0, b_gate_0, W_cand_0, b_cand_0,
           W_gate_1, b_gate_1, W_cand_1, b_cand_1):
    adj_bf = adj.astype(BF16)
    xin = inputs.reshape(B, N, 2).transpose(1, 0, 2).astype(BF16)
    xin = jnp.pad(xin, ((0, 0), (0, 0), (0, U - 2)))
    hs4 = hidden_state.reshape(2, B, N, U)
    y0_l0, h1p = _prep(xin, hs4)

    zU = jnp.zeros((3, U, U), F32)
    wgx0, wgh0 = _prep_w(W_gate_0, 2)
    wg0 = jnp.concatenate([wgx0, wgh0], axis=1).astype(BF16)
    wcx0, wcs0 = _prep_w(W_cand_0, 2)
    wcx0 = jnp.concatenate([wcx0, zU], axis=1).astype(BF16)
    wcs0 = jnp.concatenate([wcs0, zU], axis=1).astype(BF16)
    wgx1, wgh1 = _prep_w(W_gate_1, U)
    wg1 = jnp.concatenate([wgx1, wgh1], axis=1).astype(BF16)
    wcx1, wcs1 = _prep_w(W_cand_1, U)
    wcx1 = jnp.concatenate([wcx1, zU], axis=1).astype(BF16)
    wcs1 = jnp.concatenate([wcs1, zU], axis=1).astype(BF16)
    bg0 = b_gate_0.reshape(1, C)
    bc0 = b_cand_0.reshape(1, U)
    bg1 = b_gate_1.reshape(1, C)
    bc1 = b_cand_1.reshape(1, U)

    y0_l1, out0 = _layer(adj_bf, y0_l0, h1p, wg0, bg0, wcx0, wcs0, bc0)
    _, out1 = _layer(adj_bf, y0_l1, h1p, wg1, bg1, wcx1, wcs1, bc1)

    return (out1, jnp.stack([out0, out1]))


# speedup vs baseline: 3.0932x; 1.0403x over previous
import jax
import jax.numpy as jnp
from jax.experimental import pallas as pl

N = 1024
B = 32
U = 64
C = 2 * U
NB = N * B
BLK = 256
F32 = jnp.float32
BF16 = jnp.bfloat16


def _unpack_nm(h2, blkp):
    v = h2.reshape(B, blkp // 2, 2 * U)
    w = jnp.stack([v[:, :, :U], v[:, :, U:]], axis=2)
    return jnp.transpose(w.reshape(B, blkp, U), (1, 0, 2))


def _prep_body(x_ref, hs_ref, y0_ref, h1p_ref):
    x = x_ref[...]
    blkp = x.shape[0]
    hs = hs_ref[...]
    h0 = _unpack_nm(hs[0], blkp)
    h1 = _unpack_nm(hs[1], blkp)
    y0 = jnp.concatenate([x, h0.astype(BF16)], axis=-1)
    y0_ref[...] = y0.reshape(blkp, B * C)
    h1p = jnp.concatenate([jnp.zeros_like(x), h1.astype(BF16)], axis=-1)
    h1p_ref[...] = h1p.reshape(blkp, B * C)


def _prep(xin, hs):
    blkp = 128
    return pl.pallas_call(
        _prep_body,
        grid=(N // blkp,),
        in_specs=[
            pl.BlockSpec((blkp, B, U), lambda i: (i, 0, 0)),
            pl.BlockSpec((2, B, blkp * U), lambda i: (0, 0, i)),
        ],
        out_specs=[
            pl.BlockSpec((blkp, B * C), lambda i: (i, 0)),
            pl.BlockSpec((blkp, B * C), lambda i: (i, 0)),
        ],
        out_shape=[
            jax.ShapeDtypeStruct((N, B * C), BF16),
            jax.ShapeDtypeStruct((N, B * C), BF16),
        ],
    )(xin, hs)


def _d1_body(a_ref, y0_ref, y1_ref):
    y1_ref[...] = jnp.dot(a_ref[...], y0_ref[...], preferred_element_type=F32).astype(BF16)


def _d1(adj, y0):
    return pl.pallas_call(
        _d1_body,
        grid=(N // BLK,),
        in_specs=[
            pl.BlockSpec((BLK, N), lambda i: (i, 0)),
            pl.BlockSpec((N, B * C), lambda i: (0, 0)),
        ],
        out_specs=pl.BlockSpec((BLK, B * C), lambda i: (i, 0)),
        out_shape=jax.ShapeDtypeStruct((N, B * C), BF16),
    )(adj, y0)


def _gate_body(a_ref, y0_ref, y1_ref, wg_ref, bg_ref, y2_ref, s0_ref, g_ref):
    a = a_ref[...]
    y0b = y0_ref[...]
    y1b = y1_ref[pl.ds(pl.program_id(0) * BLK, BLK), :]
    y2 = 2.0 * jnp.dot(a, y1_ref[...], preferred_element_type=F32) - y0b.astype(F32)
    y2bf = y2.astype(BF16)
    y2_ref[...] = y2bf

    y0r = y0b.reshape(BLK * B, C)
    y1r = y1b.reshape(BLK * B, C)
    y2r = y2bf.reshape(BLK * B, C)
    g = jnp.dot(y0r, wg_ref[0], preferred_element_type=F32)
    g += jnp.dot(y1r, wg_ref[1], preferred_element_type=F32)
    g += jnp.dot(y2r, wg_ref[2], preferred_element_type=F32)
    g = jax.nn.sigmoid(g + bg_ref[...])
    r = g[:, :U]
    h = y0r[:, U:].astype(F32)
    s0 = (r * h).astype(BF16)
    z = jnp.zeros_like(s0)
    s0_ref[...] = jnp.concatenate([s0, z], axis=-1).reshape(BLK, B * C)
    g_ref[...] = g.astype(BF16).reshape(BLK, B * C)


def _gate(adj, y0, y1, wg, bg):
    return pl.pallas_call(
        _gate_body,
        grid=(N // BLK,),
        in_specs=[
            pl.BlockSpec((BLK, N), lambda i: (i, 0)),
            pl.BlockSpec((BLK, B * C), lambda i: (i, 0)),
            pl.BlockSpec((N, B * C), lambda i: (0, 0)),
            pl.BlockSpec((3, C, C), lambda i: (0, 0, 0)),
            pl.BlockSpec((1, C), lambda i: (0, 0)),
        ],
        out_specs=[
            pl.BlockSpec((BLK, B * C), lambda i: (i, 0)),
            pl.BlockSpec((BLK, B * C), lambda i: (i, 0)),
            pl.BlockSpec((BLK, B * C), lambda i: (i, 0)),
        ],
        out_shape=[
            jax.ShapeDtypeStruct((N, B * C), BF16),
            jax.ShapeDtypeStruct((N, B * C), BF16),
            jax.ShapeDtypeStruct((N, B * C), BF16),
        ],
    )(adj, y0, y1, wg, bg)


def _cand_body(*refs):
    (a_ref, y0_ref, y1_ref, y2_ref, s0_ref, s1_ref, g_ref, h1p_ref,
     wcx_ref, wcs_ref, bc_ref) = refs[:11]
    rest = refs[11:]
    if len(rest) == 2:
        y0n_ref, stack_ref = rest
        out1_ref = None
    else:
        _, y0n_ref, stack_ref, out1_ref = rest
    blk = a_ref.shape[0]
    a = a_ref[...]
    s0b = s0_ref[...]
    s1b = s1_ref[pl.ds(pl.program_id(0) * blk, blk), :]
    s2 = 2.0 * jnp.dot(a, s1_ref[...], preferred_element_type=F32) - s0b.astype(F32)

    y0r = y0_ref[...].reshape(blk * B, C)
    y1r = y1_ref[...].reshape(blk * B, C)
    y2r = y2_ref[...].reshape(blk * B, C)
    s0r = s0b.reshape(blk * B, C)
    s1r = s1b.reshape(blk * B, C)
    s2r = s2.astype(BF16).reshape(blk * B, C)
    c = jnp.dot(y0r, wcx_ref[0], preferred_element_type=F32)
    c += jnp.dot(y1r, wcx_ref[1], preferred_element_type=F32)
    c += jnp.dot(y2r, wcx_ref[2], preferred_element_type=F32)
    c += jnp.dot(s0r, wcs_ref[0], preferred_element_type=F32)
    c += jnp.dot(s1r, wcs_ref[1], preferred_element_type=F32)
    c += jnp.dot(s2r, wcs_ref[2], preferred_element_type=F32)
    c = jnp.tanh(c + bc_ref[...])
    gr = g_ref[...].reshape(blk * B, C)
    u = gr[:, U:].astype(F32)
    hx = y0r[:, U:].astype(F32)
    hn = u * hx + (1.0 - u) * c

    h1pr = h1p_ref[...].reshape(blk * B, C)
    y0n = jnp.concatenate([hn.astype(BF16), h1pr[:, U:]], axis=-1)
    y0n_ref[...] = y0n.reshape(blk, B * C)

    hp = hn.reshape(blk // 2, 2, B, U)
    cc = jnp.concatenate([hp[:, 0], hp[:, 1]], axis=-1)
    hbm = jnp.transpose(cc, (1, 0, 2)).reshape(B, blk * U)
    stack_ref[...] = hbm.reshape(1, B, blk * U)
    if out1_ref is not None:
        out1_ref[...] = hbm


def _cand(adj, y0, y1, y2, s0, s1, g, h1p, wcx, wcs, bc, stack_in=None):
    cblk = 128
    in_specs = [
        pl.BlockSpec((cblk, N), lambda i: (i, 0)),
        pl.BlockSpec((cblk, B * C), lambda i: (i, 0)),
        pl.BlockSpec((cblk, B * C), lambda i: (i, 0)),
        pl.BlockSpec((cblk, B * C), lambda i: (i, 0)),
        pl.BlockSpec((cblk, B * C), lambda i: (i, 0)),
        pl.BlockSpec((N, B * C), lambda i: (0, 0)),
        pl.BlockSpec((cblk, B * C), lambda i: (i, 0)),
        pl.BlockSpec((cblk, B * C), lambda i: (i, 0)),
        pl.BlockSpec((3, C, U), lambda i: (0, 0, 0)),
        pl.BlockSpec((3, C, U), lambda i: (0, 0, 0)),
        pl.BlockSpec((1, U), lambda i: (0, 0)),
    ]
    slot = 0 if stack_in is None else 1
    out_specs = [
        pl.BlockSpec((cblk, B * C), lambda i: (i, 0)),
        pl.BlockSpec((1, B, cblk * U), lambda i: (slot, 0, i)),
    ]
    out_shape = [
        jax.ShapeDtypeStruct((N, B * C), BF16),
        jax.ShapeDtypeStruct((2, B, N * U), F32),
    ]
    args = [adj, y0, y1, y2, s0, s1, g, h1p, wcx, wcs, bc]
    aliases = {}
    if stack_in is not None:
        in_specs.append(pl.BlockSpec((1, 8, 128), lambda i: (0, 0, 0)))
        args.append(stack_in)
        aliases = {11: 1}
        out_specs.append(pl.BlockSpec((B, cblk * U), lambda i: (0, i)))
        out_shape.append(jax.ShapeDtypeStruct((B, N * U), F32))
    return pl.pallas_call(
        _cand_body,
        grid=(N // cblk,),
        in_specs=in_specs,
        out_specs=out_specs,
        out_shape=out_shape,
        input_output_aliases=aliases,
    )(*args)


def _layer(adj_bf, y0, h1p, wg, bg, wcx, wcs, bc, stack_in=None):
    y1 = _d1(adj_bf, y0)
    y2, s0, g = _gate(adj_bf, y0, y1, wg, bg)
    s1 = _d1(adj_bf, s0)
    return _cand(adj_bf, y0, y1, y2, s0, s1, g, h1p, wcx, wcs, bc, stack_in)


def _prep_w(w, cin):
    o = w.shape[1]
    wr = w.reshape(cin + U, 3, o).transpose(1, 0, 2)
    wx = wr[:, :cin, :]
    wh = wr[:, cin:, :]
    pad = jnp.zeros((3, U - cin, o), w.dtype)
    wxp = jnp.concatenate([wx, pad], axis=1)
    return wxp, wh


def kernel(inputs, adj, hidden_state,
           W_gate_0, b_gate_0, W_cand_0, b_cand_0,
           W_gate_1, b_gate_1, W_cand_1, b_cand_1):
    adj_bf = adj.astype(BF16)
    xin = inputs.reshape(B, N, 2).transpose(1, 0, 2).astype(BF16)
    xin = jnp.pad(xin, ((0, 0), (0, 0), (0, U - 2)))
    y0_l0, h1p = _prep(xin, hidden_state)

    zU = jnp.zeros((3, U, U), F32)
    wgx0, wgh0 = _prep_w(W_gate_0, 2)
    wg0 = jnp.concatenate([wgx0, wgh0], axis=1).astype(BF16)
    wcx0, wcs0 = _prep_w(W_cand_0, 2)
    wcx0 = jnp.concatenate([wcx0, zU], axis=1).astype(BF16)
    wcs0 = jnp.concatenate([wcs0, zU], axis=1).astype(BF16)
    wgx1, wgh1 = _prep_w(W_gate_1, U)
    wg1 = jnp.concatenate([wgx1, wgh1], axis=1).astype(BF16)
    wcx1, wcs1 = _prep_w(W_cand_1, U)
    wcx1 = jnp.concatenate([wcx1, zU], axis=1).astype(BF16)
    wcs1 = jnp.concatenate([wcs1, zU], axis=1).astype(BF16)
    bg0 = b_gate_0.reshape(1, C)
    bc0 = b_cand_0.reshape(1, U)
    bg1 = b_gate_1.reshape(1, C)
    bc1 = b_cand_1.reshape(1, U)

    y0_l1, stack0 = _layer(adj_bf, y0_l0, h1p, wg0, bg0, wcx0, wcs0, bc0)
    _, stack, out1 = _layer(adj_bf, y0_l1, h1p, wg1, bg1, wcx1, wcs1, bc1,
                            stack_in=stack0)

    return (out1, stack)


# speedup vs baseline: 3.1957x; 1.0331x over previous
import functools

import jax
import jax.numpy as jnp
from jax.experimental import pallas as pl

N = 1024
B = 32
U = 64
C = 2 * U
NB = N * B
BLK = 256
F32 = jnp.float32
BF16 = jnp.bfloat16


def _prep_body(x_ref, hs_ref, y0_ref, h1p_ref):
    x = x_ref[...]
    blkp = x.shape[0]
    h0 = jnp.transpose(hs_ref[0], (1, 0, 2))
    h1 = jnp.transpose(hs_ref[1], (1, 0, 2))
    y0 = jnp.concatenate([x, h0], axis=-1)
    y0_ref[...] = y0.reshape(blkp, B * C)
    h1p = jnp.concatenate([jnp.zeros_like(x), h1], axis=-1)
    h1p_ref[...] = h1p.reshape(blkp, B * C)


def _prep(xin, hs):
    blkp = 128
    return pl.pallas_call(
        _prep_body,
        grid=(N // blkp,),
        in_specs=[
            pl.BlockSpec((blkp, B, U), lambda i: (i, 0, 0)),
            pl.BlockSpec((2, B, blkp, U), lambda i: (0, 0, i, 0)),
        ],
        out_specs=[
            pl.BlockSpec((blkp, B * C), lambda i: (i, 0)),
            pl.BlockSpec((blkp, B * C), lambda i: (i, 0)),
        ],
        out_shape=[
            jax.ShapeDtypeStruct((N, B * C), BF16),
            jax.ShapeDtypeStruct((N, B * C), BF16),
        ],
    )(xin, hs)


def _d1_body(a_ref, y0_ref, y1_ref):
    y1_ref[...] = jnp.dot(a_ref[...], y0_ref[...], preferred_element_type=F32).astype(BF16)


def _d1(adj, y0):
    return pl.pallas_call(
        _d1_body,
        grid=(N // BLK,),
        in_specs=[
            pl.BlockSpec((BLK, N), lambda i: (i, 0)),
            pl.BlockSpec((N, B * C), lambda i: (0, 0)),
        ],
        out_specs=pl.BlockSpec((BLK, B * C), lambda i: (i, 0)),
        out_shape=jax.ShapeDtypeStruct((N, B * C), BF16),
    )(adj, y0)


def _gate_body(a_ref, y0_ref, y1_ref, wg_ref, bg_ref, y2_ref, s0_ref, g_ref):
    a = a_ref[...]
    y0b = y0_ref[...]
    y1b = y1_ref[pl.ds(pl.program_id(0) * BLK, BLK), :]
    y2 = 2.0 * jnp.dot(a, y1_ref[...], preferred_element_type=F32) - y0b.astype(F32)
    y2bf = y2.astype(BF16)
    y2_ref[...] = y2bf

    y0r = y0b.reshape(BLK * B, C)
    y1r = y1b.reshape(BLK * B, C)
    y2r = y2bf.reshape(BLK * B, C)
    g = jnp.dot(y0r, wg_ref[0], preferred_element_type=F32)
    g += jnp.dot(y1r, wg_ref[1], preferred_element_type=F32)
    g += jnp.dot(y2r, wg_ref[2], preferred_element_type=F32)
    g = jax.nn.sigmoid(g + bg_ref[...])
    r = g[:, :U]
    h = y0r[:, U:].astype(F32)
    s0 = (r * h).astype(BF16)
    z = jnp.zeros_like(s0)
    s0_ref[...] = jnp.concatenate([s0, z], axis=-1).reshape(BLK, B * C)
    g_ref[...] = g.astype(BF16).reshape(BLK, B * C)


def _gate(adj, y0, y1, wg, bg):
    return pl.pallas_call(
        _gate_body,
        grid=(N // BLK,),
        in_specs=[
            pl.BlockSpec((BLK, N), lambda i: (i, 0)),
            pl.BlockSpec((BLK, B * C), lambda i: (i, 0)),
            pl.BlockSpec((N, B * C), lambda i: (0, 0)),
            pl.BlockSpec((3, C, C), lambda i: (0, 0, 0)),
            pl.BlockSpec((1, C), lambda i: (0, 0)),
        ],
        out_specs=[
            pl.BlockSpec((BLK, B * C), lambda i: (i, 0)),
            pl.BlockSpec((BLK, B * C), lambda i: (i, 0)),
            pl.BlockSpec((BLK, B * C), lambda i: (i, 0)),
        ],
        out_shape=[
            jax.ShapeDtypeStruct((N, B * C), BF16),
            jax.ShapeDtypeStruct((N, B * C), BF16),
            jax.ShapeDtypeStruct((N, B * C), BF16),
        ],
    )(adj, y0, y1, wg, bg)


def _cand_body(is_l0, *refs):
    (a_ref, y0_ref, y1_ref, y2_ref, s0_ref, s1_ref, g_ref,
     wcx_ref, wcs_ref, bc_ref) = refs[:10]
    if is_l0:
        h1p_ref, y0n_ref, stack_ref = refs[10:]
        out1_ref = None
    else:
        _, stack_ref, out1_ref = refs[10:]
        h1p_ref = y0n_ref = None
    blk = a_ref.shape[0]
    a = a_ref[...]
    s0b = s0_ref[...]
    s1b = s1_ref[pl.ds(pl.program_id(0) * blk, blk), :]
    s2 = 2.0 * jnp.dot(a, s1_ref[...], preferred_element_type=F32) - s0b.astype(F32)

    y0r = y0_ref[...].reshape(blk * B, C)
    y1r = y1_ref[...].reshape(blk * B, C)
    y2r = y2_ref[...].reshape(blk * B, C)
    s0r = s0b.reshape(blk * B, C)
    s1r = s1b.reshape(blk * B, C)
    s2r = s2.astype(BF16).reshape(blk * B, C)
    c = jnp.dot(y0r, wcx_ref[0], preferred_element_type=F32)
    c += jnp.dot(y1r, wcx_ref[1], preferred_element_type=F32)
    c += jnp.dot(y2r, wcx_ref[2], preferred_element_type=F32)
    c += jnp.dot(s0r, wcs_ref[0], preferred_element_type=F32)
    c += jnp.dot(s1r, wcs_ref[1], preferred_element_type=F32)
    c += jnp.dot(s2r, wcs_ref[2], preferred_element_type=F32)
    c = jnp.tanh(c + bc_ref[...])
    gr = g_ref[...].reshape(blk * B, C)
    u = gr[:, U:].astype(F32)
    hx = y0r[:, U:].astype(F32)
    hn = u * hx + (1.0 - u) * c

    if y0n_ref is not None:
        h1pr = h1p_ref[...].reshape(blk * B, C)
        y0n = jnp.concatenate([hn.astype(BF16), h1pr[:, U:]], axis=-1)
        y0n_ref[...] = y0n.reshape(blk, B * C)

    hp = hn.reshape(blk // 2, 2, B, U)
    cc = jnp.concatenate([hp[:, 0], hp[:, 1]], axis=-1)
    hbm = jnp.transpose(cc, (1, 0, 2)).reshape(B, blk * U)
    stack_ref[...] = hbm.reshape(1, B, blk * U)
    if out1_ref is not None:
        out1_ref[...] = hbm


def _cand(adj, y0, y1, y2, s0, s1, g, h1p, wcx, wcs, bc, stack_in=None):
    cblk = 128
    is_l0 = stack_in is None
    in_specs = [
        pl.BlockSpec((cblk, N), lambda i: (i, 0)),
        pl.BlockSpec((cblk, B * C), lambda i: (i, 0)),
        pl.BlockSpec((cblk, B * C), lambda i: (i, 0)),
        pl.BlockSpec((cblk, B * C), lambda i: (i, 0)),
        pl.BlockSpec((cblk, B * C), lambda i: (i, 0)),
        pl.BlockSpec((N, B * C), lambda i: (0, 0)),
        pl.BlockSpec((cblk, B * C), lambda i: (i, 0)),
        pl.BlockSpec((3, C, U), lambda i: (0, 0, 0)),
        pl.BlockSpec((3, C, U), lambda i: (0, 0, 0)),
        pl.BlockSpec((1, U), lambda i: (0, 0)),
    ]
    args = [adj, y0, y1, y2, s0, s1, g, wcx, wcs, bc]
    slot = 0 if is_l0 else 1
    stack_spec = pl.BlockSpec((1, B, cblk * U), lambda i: (slot, 0, i))
    stack_shape = jax.ShapeDtypeStruct((2, B, N * U), F32)
    aliases = {}
    if is_l0:
        in_specs.append(pl.BlockSpec((cblk, B * C), lambda i: (i, 0)))
        args.append(h1p)
        out_specs = [pl.BlockSpec((cblk, B * C), lambda i: (i, 0)), stack_spec]
        out_shape = [jax.ShapeDtypeStruct((N, B * C), BF16), stack_shape]
    else:
        in_specs.append(pl.BlockSpec((1, 8, 128), lambda i: (0, 0, 0)))
        args.append(stack_in)
        aliases = {10: 0}
        out_specs = [stack_spec, pl.BlockSpec((B, cblk * U), lambda i: (0, i))]
        out_shape = [stack_shape, jax.ShapeDtypeStruct((B, N * U), F32)]
    return pl.pallas_call(
        functools.partial(_cand_body, is_l0),
        grid=(N // cblk,),
        in_specs=in_specs,
        out_specs=out_specs,
        out_shape=out_shape,
        input_output_aliases=aliases,
    )(*args)


def _layer(adj_bf, y0, h1p, wg, bg, wcx, wcs, bc, stack_in=None):
    y1 = _d1(adj_bf, y0)
    y2, s0, g = _gate(adj_bf, y0, y1, wg, bg)
    s1 = _d1(adj_bf, s0)
    return _cand(adj_bf, y0, y1, y2, s0, s1, g, h1p, wcx, wcs, bc, stack_in)


def _prep_w(w, cin):
    o = w.shape[1]
    wr = w.reshape(cin + U, 3, o).transpose(1, 0, 2)
    wx = wr[:, :cin, :]
    wh = wr[:, cin:, :]
    pad = jnp.zeros((3, U - cin, o), w.dtype)
    wxp = jnp.concatenate([wx, pad], axis=1)
    return wxp, wh


def kernel(inputs, adj, hidden_state,
           W_gate_0, b_gate_0, W_cand_0, b_cand_0,
           W_gate_1, b_gate_1, W_cand_1, b_cand_1):
    adj_bf = adj.astype(BF16)
    xin = inputs.astype(BF16).reshape(B, N, 2).transpose(1, 0, 2)
    xin = jnp.pad(xin, ((0, 0), (0, 0), (0, U - 2)))
    hsb = hidden_state.astype(BF16).reshape(2, B, N, U)
    y0_l0, h1p = _prep(xin, hsb)

    zU = jnp.zeros((3, U, U), F32)
    wgx0, wgh0 = _prep_w(W_gate_0, 2)
    wg0 = jnp.concatenate([wgx0, wgh0], axis=1).astype(BF16)
    wcx0, wcs0 = _prep_w(W_cand_0, 2)
    wcx0 = jnp.concatenate([wcx0, zU], axis=1).astype(BF16)
    wcs0 = jnp.concatenate([wcs0, zU], axis=1).astype(BF16)
    wgx1, wgh1 = _prep_w(W_gate_1, U)
    wg1 = jnp.concatenate([wgx1, wgh1], axis=1).astype(BF16)
    wcx1, wcs1 = _prep_w(W_cand_1, U)
    wcx1 = jnp.concatenate([wcx1, zU], axis=1).astype(BF16)
    wcs1 = jnp.concatenate([wcs1, zU], axis=1).astype(BF16)
    bg0 = b_gate_0.reshape(1, C)
    bc0 = b_cand_0.reshape(1, U)
    bg1 = b_gate_1.reshape(1, C)
    bc1 = b_cand_1.reshape(1, U)

    y0_l1, stack0 = _layer(adj_bf, y0_l0, h1p, wg0, bg0, wcx0, wcs0, bc0)
    stack, out1 = _layer(adj_bf, y0_l1, h1p, wg1, bg1, wcx1, wcs1, bc1,
                         stack_in=stack0)

    return (out1, stack)


# speedup vs baseline: 3.4105x; 1.0672x over previous
import functools

import jax
import jax.numpy as jnp
from jax.experimental import pallas as pl
from jax.experimental.pallas import tpu as pltpu

N = 1024
B = 32
U = 64
C = 2 * U
BLK = 256
F32 = jnp.float32
BF16 = jnp.bfloat16

_ARB2 = pltpu.CompilerParams(dimension_semantics=("arbitrary", "arbitrary"))


def _prep_body(x_ref, hs_ref, y0_ref, h1p_ref):
    x = x_ref[...]
    blkp = x.shape[0]
    h0 = jnp.transpose(hs_ref[0], (1, 0, 2))
    h1 = jnp.transpose(hs_ref[1], (1, 0, 2))
    y0 = jnp.concatenate([x, h0], axis=-1)
    y0_ref[...] = y0.reshape(blkp, B * C)
    h1p = jnp.concatenate([jnp.zeros_like(x), h1], axis=-1)
    h1p_ref[...] = h1p.reshape(blkp, B * C)


def _prep(xin, hs):
    blkp = 128
    return pl.pallas_call(
        _prep_body,
        grid=(N // blkp,),
        in_specs=[
            pl.BlockSpec((blkp, B, U), lambda i: (i, 0, 0)),
            pl.BlockSpec((2, B, blkp, U), lambda i: (0, 0, i, 0)),
        ],
        out_specs=[
            pl.BlockSpec((blkp, B * C), lambda i: (i, 0)),
            pl.BlockSpec((blkp, B * C), lambda i: (i, 0)),
        ],
        out_shape=[
            jax.ShapeDtypeStruct((N, B * C), BF16),
            jax.ShapeDtypeStruct((N, B * C), BF16),
        ],
    )(xin, hs)


def _gate_body(a_ref, y0_ref, wg_ref, wcx_ref, bg_ref, s0p_ref, p_ref, y1_scr):
    ph = pl.program_id(0)
    i = pl.program_id(1)
    a = a_ref[...]

    @pl.when(ph == 0)
    def _():
        y1_scr[pl.ds(i * BLK, BLK), :] = jnp.dot(
            a, y0_ref[...], preferred_element_type=F32).astype(BF16)

    @pl.when(ph == 1)
    def _():
        rows = pl.ds(i * BLK, BLK)
        y0b = y0_ref[rows, :]
        y1b = y1_scr[rows, :]
        y2 = 2.0 * jnp.dot(a, y1_scr[...], preferred_element_type=F32) - y0b.astype(F32)
        y0r = y0b.reshape(BLK * B, C)
        y1r = y1b.reshape(BLK * B, C)
        y2r = y2.astype(BF16).reshape(BLK * B, C)
        g = jnp.dot(y0r, wg_ref[0], preferred_element_type=F32)
        g += jnp.dot(y1r, wg_ref[1], preferred_element_type=F32)
        g += jnp.dot(y2r, wg_ref[2], preferred_element_type=F32)
        g = jax.nn.sigmoid(g + bg_ref[...])
        cx = jnp.dot(y0r, wcx_ref[0], preferred_element_type=F32)
        cx += jnp.dot(y1r, wcx_ref[1], preferred_element_type=F32)
        cx += jnp.dot(y2r, wcx_ref[2], preferred_element_type=F32)
        r = g[:, :U]
        u = g[:, U:]
        hx = y0r[:, U:].astype(F32)
        s0 = (r * hx).astype(BF16)
        s0p_ref[...] = jnp.concatenate([s0, jnp.zeros_like(s0)], axis=-1).reshape(BLK, B * C)
        p_ref[...] = jnp.concatenate(
            [cx.astype(BF16), u.astype(BF16)], axis=-1).reshape(BLK, B * C)


def _gate(adj, y0, wg, wcx, bg):
    return pl.pallas_call(
        _gate_body,
        grid=(2, N // BLK),
        in_specs=[
            pl.BlockSpec((BLK, N), lambda p, i: (i, 0)),
            pl.BlockSpec((N, B * C), lambda p, i: (0, 0)),
            pl.BlockSpec((3, C, C), lambda p, i: (0, 0, 0)),
            pl.BlockSpec((3, C, U), lambda p, i: (0, 0, 0)),
            pl.BlockSpec((1, C), lambda p, i: (0, 0)),
        ],
        out_specs=[
            pl.BlockSpec((BLK, B * C), lambda p, i: (jnp.where(p == 1, i, 0), 0)),
            pl.BlockSpec((BLK, B * C), lambda p, i: (jnp.where(p == 1, i, 0), 0)),
        ],
        out_shape=[
            jax.ShapeDtypeStruct((N, B * C), BF16),
            jax.ShapeDtypeStruct((N, B * C), BF16),
        ],
        scratch_shapes=[pltpu.VMEM((N, B * C), BF16)],
        compiler_params=_ARB2,
    )(adj, y0, wg, wcx, bg)


def _cand_body(is_l0, *refs):
    (a_ref, s0p_ref, p_ref, y0_ref, wcs_ref, bc_ref) = refs[:6]
    if is_l0:
        h1p_ref, y0n_ref, stack_ref = refs[6:9]
        out1_ref = None
        s1_scr = refs[9]
    else:
        _, stack_ref, out1_ref = refs[6:9]
        h1p_ref = y0n_ref = None
        s1_scr = refs[9]
    ph = pl.program_id(0)
    i = pl.program_id(1)
    a = a_ref[...]

    @pl.when(ph == 0)
    def _():
        s1_scr[pl.ds(i * BLK, BLK), :] = jnp.dot(
            a, s0p_ref[...], preferred_element_type=F32).astype(BF16)

    @pl.when(ph == 1)
    def _():
        rows = pl.ds(i * BLK, BLK)
        s0b = s0p_ref[rows, :]
        s1b = s1_scr[rows, :]
        s2 = 2.0 * jnp.dot(a, s1_scr[...], preferred_element_type=F32) - s0b.astype(F32)
        s0r = s0b.reshape(BLK * B, C)
        s1r = s1b.reshape(BLK * B, C)
        s2r = s2.astype(BF16).reshape(BLK * B, C)
        pr = p_ref[...].reshape(BLK * B, C)
        c = jnp.dot(s0r, wcs_ref[0], preferred_element_type=F32)
        c += jnp.dot(s1r, wcs_ref[1], preferred_element_type=F32)
        c += jnp.dot(s2r, wcs_ref[2], preferred_element_type=F32)
        c = jnp.tanh(c + pr[:, :U].astype(F32) + bc_ref[...])
        u = pr[:, U:].astype(F32)
        y0r = y0_ref[...].reshape(BLK * B, C)
        hx = y0r[:, U:].astype(F32)
        hn = u * hx + (1.0 - u) * c

        if y0n_ref is not None:
            h1pr = h1p_ref[...].reshape(BLK * B, C)
            y0n = jnp.concatenate([hn.astype(BF16), h1pr[:, U:]], axis=-1)
            y0n_ref[...] = y0n.reshape(BLK, B * C)

        hp = hn.reshape(BLK // 2, 2, B, U)
        cc = jnp.concatenate([hp[:, 0], hp[:, 1]], axis=-1)
        hbm = jnp.transpose(cc, (1, 0, 2)).reshape(B, BLK * U)
        stack_ref[...] = hbm.reshape(1, B, BLK * U)
        if out1_ref is not None:
            out1_ref[...] = hbm


def _cand(adj, s0p, p, y0, wcs, bc, h1p, stack_in):
    is_l0 = stack_in is None

    def blk1(p_, i_):
        return (jnp.where(p_ == 1, i_, 0), 0)

    in_specs = [
        pl.BlockSpec((BLK, N), lambda p_, i_: (i_, 0)),
        pl.BlockSpec((N, B * C), lambda p_, i_: (0, 0)),
        pl.BlockSpec((BLK, B * C), blk1),
        pl.BlockSpec((BLK, B * C), blk1),
        pl.BlockSpec((3, C, U), lambda p_, i_: (0, 0, 0)),
        pl.BlockSpec((1, U), lambda p_, i_: (0, 0)),
    ]
    args = [adj, s0p, p, y0, wcs, bc]
    slot = 0 if is_l0 else 1
    stack_spec = pl.BlockSpec(
        (1, B, BLK * U), lambda p_, i_: (slot, 0, jnp.where(p_ == 1, i_, 0)))
    stack_shape = jax.ShapeDtypeStruct((2, B, N * U), F32)
    aliases = {}
    if is_l0:
        in_specs.append(pl.BlockSpec((BLK, B * C), blk1))
        args.append(h1p)
        out_specs = [pl.BlockSpec((BLK, B * C), blk1), stack_spec]
        out_shape = [jax.ShapeDtypeStruct((N, B * C), BF16), stack_shape]
    else:
        in_specs.append(pl.BlockSpec((1, 8, 128), lambda p_, i_: (0, 0, 0)))
        args.append(stack_in)
        aliases = {6: 0}
        out_specs = [
            stack_spec,
            pl.BlockSpec((B, BLK * U), lambda p_, i_: (0, jnp.where(p_ == 1, i_, 0))),
        ]
        out_shape = [stack_shape, jax.ShapeDtypeStruct((B, N * U), F32)]
    return pl.pallas_call(
        functools.partial(_cand_body, is_l0),
        grid=(2, N // BLK),
        in_specs=in_specs,
        out_specs=out_specs,
        out_shape=out_shape,
        input_output_aliases=aliases,
        scratch_shapes=[pltpu.VMEM((N, B * C), BF16)],
        compiler_params=_ARB2,
    )(*args)


def _layer(adj_bf, y0, h1p, wg, bg, wcx, wcs, bc, stack_in=None):
    s0p, p = _gate(adj_bf, y0, wg, wcx, bg)
    return _cand(adj_bf, s0p, p, y0, wcs, bc, h1p, stack_in)


def _prep_w(w, cin):
    o = w.shape[1]
    wr = w.reshape(cin + U, 3, o).transpose(1, 0, 2)
    wx = wr[:, :cin, :]
    wh = wr[:, cin:, :]
    pad = jnp.zeros((3, U - cin, o), w.dtype)
    wxp = jnp.concatenate([wx, pad], axis=1)
    return wxp, wh


def kernel(inputs, adj, hidden_state,
           W_gate_0, b_gate_0, W_cand_0, b_cand_0,
           W_gate_1, b_gate_1, W_cand_1, b_cand_1):
    adj_bf = adj.astype(BF16)
    xin = inputs.astype(BF16).reshape(B, N, 2).transpose(1, 0, 2)
    xin = jnp.pad(xin, ((0, 0), (0, 0), (0, U - 2)))
    hsb = hidden_state.astype(BF16).reshape(2, B, N, U)
    y0_l0, h1p = _prep(xin, hsb)

    zU = jnp.zeros((3, U, U), F32)
    wgx0, wgh0 = _prep_w(W_gate_0, 2)
    wg0 = jnp.concatenate([wgx0, wgh0], axis=1).astype(BF16)
    wcx0, wcs0 = _prep_w(W_cand_0, 2)
    wcx0 = jnp.concatenate([wcx0, zU], axis=1).astype(BF16)
    wcs0 = jnp.concatenate([wcs0, zU], axis=1).astype(BF16)
    wgx1, wgh1 = _prep_w(W_gate_1, U)
    wg1 = jnp.concatenate([wgx1, wgh1], axis=1).astype(BF16)
    wcx1, wcs1 = _prep_w(W_cand_1, U)
    wcx1 = jnp.concatenate([wcx1, zU], axis=1).astype(BF16)
    wcs1 = jnp.concatenate([wcs1, zU], axis=1).astype(BF16)
    bg0 = b_gate_0.reshape(1, C)
    bc0 = b_cand_0.reshape(1, U)
    bg1 = b_gate_1.reshape(1, C)
    bc1 = b_cand_1.reshape(1, U)

    y0_l1, stack0 = _layer(adj_bf, y0_l0, h1p, wg0, bg0, wcx0, wcs0, bc0)
    stack, out1 = _layer(adj_bf, y0_l1, h1p, wg1, bg1, wcx1, wcs1, bc1,
                         stack_in=stack0)

    return (out1, stack)
